# baseline (speedup 1.0000x reference)
import jax
import jax.numpy as jnp
from jax import lax
from jax.experimental import pallas as pl
from jax.experimental.pallas import tpu as pltpu
from jax.experimental.pallas import tpu_sc as plsc

N = 10000
D = 128
B = 256
G = 128
H = 128

NC = 2
NS = 16
NW = NC * NS
ROWS_W = 320
LAST_ROWS = N - (NW - 1) * ROWS_W
CHUNK = 128
NCHUNK = 3
STAGE = NCHUNK * CHUNK
ACC_ROWS = 272
DUMMY = 256


def _seg_kernel_body(x_hbm, ids_hbm, zeros_hbm, sums_hbm,
                     rows_v, ids2d, acc_sh, sem_rows, sem_ids, sem_zero):
    c = lax.axis_index("c")
    s = lax.axis_index("s")
    w = c * NS + s
    base = w * ROWS_W

    @pl.when(w < NW - 1)
    def _():
        pltpu.async_copy(x_hbm.at[pl.ds(base, ROWS_W)],
                         rows_v.at[pl.ds(0, ROWS_W)], sem_rows)

    @pl.when(w == NW - 1)
    def _():
        pltpu.async_copy(x_hbm.at[pl.ds(base, LAST_ROWS)],
                         rows_v.at[pl.ds(0, LAST_ROWS)], sem_rows)

    pltpu.async_copy(ids_hbm.at[w], ids2d, sem_ids)
    pltpu.async_copy(zeros_hbm, acc_sh.at[pl.ds(s * 16, 16)], sem_zero)

    pltpu.make_async_copy(ids_hbm.at[w], ids2d, sem_ids).wait()
    pltpu.make_async_copy(zeros_hbm, acc_sh.at[pl.ds(s * 16, 16)],
                          sem_zero).wait()

    plsc.subcore_barrier()

    @pl.when(w < NW - 1)
    def _():
        pltpu.make_async_copy(x_hbm.at[pl.ds(base, ROWS_W)],
                              rows_v.at[pl.ds(0, ROWS_W)], sem_rows).wait()

    @pl.when(w == NW - 1)
    def _():
        pltpu.make_async_copy(x_hbm.at[pl.ds(base, LAST_ROWS)],
                              rows_v.at[pl.ds(0, LAST_ROWS)], sem_rows).wait()

    for j in range(NCHUNK):
        pltpu.sync_copy(rows_v.at[pl.ds(j * CHUNK, CHUNK)],
                        acc_sh.at[ids2d.at[j]], add=True)

    plsc.subcore_barrier()

    @pl.when(s == 0)
    def _():
        pltpu.sync_copy(acc_sh.at[pl.ds(0, B)], sums_hbm.at[c])


_seg_reduce = pl.kernel(
    _seg_kernel_body,
    out_type=jax.ShapeDtypeStruct((NC, B, D), jnp.float32),
    mesh=plsc.VectorSubcoreMesh(core_axis_name="c", subcore_axis_name="s"),
    scratch_types=[
        pltpu.VMEM((STAGE, D), jnp.float32),
        pltpu.VMEM((NCHUNK, CHUNK), jnp.int32),
        pltpu.VMEM_SHARED((ACC_ROWS, D), jnp.float32),
        pltpu.SemaphoreType.DMA,
        pltpu.SemaphoreType.DMA,
        pltpu.SemaphoreType.DMA,
    ],
)


def _hist_body(ids_ref, cnt_ref):
    segs = lax.broadcasted_iota(jnp.int32, (B, 1), 0)

    def step(r, acc):
        row = ids_ref[pl.ds(r, 1), :]
        return acc + (row == segs).astype(jnp.float32)

    acc = lax.fori_loop(0, NW * NCHUNK, step,
                        jnp.zeros((B, 128), jnp.float32))
    cnt_ref[...] = jnp.sum(acc, axis=1, keepdims=True)


def _hist(ids_pad):
    return pl.pallas_call(
        _hist_body,
        out_shape=jax.ShapeDtypeStruct((B, 1), jnp.float32),
    )(ids_pad)


def _mlp_body(sums_ref, cnt_ref, u_ref, W1_ref, b1_ref, W2_ref, b2_ref,
              out_ref):
    seg = sums_ref[0] + sums_ref[1]
    cnt = cnt_ref[...]
    mean = seg / jnp.maximum(cnt, 1.0)
    u = u_ref[...]
    h = (lax.dot_general(u, W1_ref[0:G, :], (((1,), (0,)), ((), ())),
                         preferred_element_type=jnp.float32)
         + lax.dot_general(mean, W1_ref[G:G + D, :], (((1,), (0,)), ((), ())),
                           preferred_element_type=jnp.float32)
         + b1_ref[...])
    h = jnp.maximum(h, 0.0)
    out_ref[...] = (lax.dot_general(h, W2_ref[...], (((1,), (0,)), ((), ())),
                                    preferred_element_type=jnp.float32)
                    + b2_ref[...] + u)


def _mlp(sums, cnt, u, W1, b1, W2, b2):
    return pl.pallas_call(
        _mlp_body,
        out_shape=jax.ShapeDtypeStruct((B, G), jnp.float32),
    )(sums, cnt, u, W1, b1.reshape(1, H), W2, b2.reshape(1, G))


@jax.jit
def kernel(x, edge_index, edge_attr, u, batch, W1, b1, W2, b2):
    ids = batch.astype(jnp.int32)
    ids_p = jnp.pad(ids, (0, NW * ROWS_W - N), constant_values=DUMMY)
    ids_p = ids_p.reshape(NW, ROWS_W)
    ids_p = jnp.pad(ids_p, ((0, 0), (0, STAGE - ROWS_W)),
                    constant_values=DUMMY)
    ids_sc = ids_p.reshape(NW, NCHUNK, CHUNK)
    zeros16 = jnp.zeros((16, D), jnp.float32)
    cnt = _hist(ids_sc.reshape(NW * NCHUNK, CHUNK))
    sums = _seg_reduce(x, ids_sc, zeros16)
    return _mlp(sums, cnt, u, W1, b1, W2, b2)


# speedup vs baseline: 1.0260x; 1.0260x over previous
import jax
import jax.numpy as jnp
from jax import lax
from jax.experimental import pallas as pl
from jax.experimental.pallas import tpu as pltpu
from jax.experimental.pallas import tpu_sc as plsc

N = 10000
D = 128
B = 256
G = 128
H = 128

NC = 2
NS = 16
NW = NC * NS
ROWS_W = 320
LAST_ROWS = N - (NW - 1) * ROWS_W
NPAD = NW * ROWS_W
ACC_ROWS = 272
HR, HC = 20, 512


def _seg_kernel_body(x_hbm, ids_hbm, zeros_hbm, sums_hbm,
                     rows_v, idsA, idsB, acc_sh, sem_rows, sem_ids, sem_zero):
    c = lax.axis_index("c")
    s = lax.axis_index("s")
    w = c * NS + s
    base = w * ROWS_W

    @pl.when(w < NW - 1)
    def _():
        pltpu.async_copy(x_hbm.at[pl.ds(base, ROWS_W)],
                         rows_v.at[pl.ds(0, ROWS_W)], sem_rows)

    @pl.when(w == NW - 1)
    def _():
        pltpu.async_copy(x_hbm.at[pl.ds(base, LAST_ROWS)],
                         rows_v.at[pl.ds(0, LAST_ROWS)], sem_rows)

    pltpu.async_copy(ids_hbm.at[pl.ds(base, 128)], idsA.at[0], sem_ids)
    pltpu.async_copy(ids_hbm.at[pl.ds(base + 128, 128)], idsA.at[1], sem_ids)
    pltpu.async_copy(ids_hbm.at[pl.ds(base + 256, 64)], idsB, sem_ids)
    pltpu.async_copy(zeros_hbm, acc_sh.at[pl.ds(s * 16, 16)], sem_zero)

    pltpu.make_async_copy(ids_hbm.at[pl.ds(base, 128)], idsA.at[0],
                          sem_ids).wait()
    pltpu.make_async_copy(ids_hbm.at[pl.ds(base + 128, 128)], idsA.at[1],
                          sem_ids).wait()
    pltpu.make_async_copy(ids_hbm.at[pl.ds(base + 256, 64)], idsB,
                          sem_ids).wait()
    pltpu.make_async_copy(zeros_hbm, acc_sh.at[pl.ds(s * 16, 16)],
                          sem_zero).wait()

    plsc.subcore_barrier()

    @pl.when(w < NW - 1)
    def _():
        pltpu.make_async_copy(x_hbm.at[pl.ds(base, ROWS_W)],
                              rows_v.at[pl.ds(0, ROWS_W)], sem_rows).wait()

    @pl.when(w == NW - 1)
    def _():
        pltpu.make_async_copy(x_hbm.at[pl.ds(base, LAST_ROWS)],
                              rows_v.at[pl.ds(0, LAST_ROWS)], sem_rows).wait()

    pltpu.sync_copy(rows_v.at[pl.ds(0, 128)], acc_sh.at[idsA.at[0]], add=True)
    pltpu.sync_copy(rows_v.at[pl.ds(128, 128)], acc_sh.at[idsA.at[1]],
                    add=True)
    pltpu.sync_copy(rows_v.at[pl.ds(256, 64)], acc_sh.at[idsB], add=True)

    plsc.subcore_barrier()

    @pl.when(s == 0)
    def _():
        pltpu.sync_copy(acc_sh.at[pl.ds(0, B)], sums_hbm.at[c])


_seg_reduce = pl.kernel(
    _seg_kernel_body,
    out_type=jax.ShapeDtypeStruct((NC, B, D), jnp.float32),
    mesh=plsc.VectorSubcoreMesh(core_axis_name="c", subcore_axis_name="s"),
    scratch_types=[
        pltpu.VMEM((ROWS_W, D), jnp.float32),
        pltpu.VMEM((2, 128), jnp.int32),
        pltpu.VMEM((64,), jnp.int32),
        pltpu.VMEM_SHARED((ACC_ROWS, D), jnp.float32),
        pltpu.SemaphoreType.DMA,
        pltpu.SemaphoreType.DMA,
        pltpu.SemaphoreType.DMA,
    ],
)


def _hist_body(ids_ref, cnt_ref):
    segs = lax.broadcasted_iota(jnp.int32, (B, 1), 0)

    def step(r, acc):
        row = ids_ref[pl.ds(r, 1), :]
        return acc + (row == segs).astype(jnp.float32)

    acc = lax.fori_loop(0, HR, step, jnp.zeros((B, HC), jnp.float32))
    a = (acc[:, 0:128] + acc[:, 128:256]
         + acc[:, 256:384] + acc[:, 384:512])
    cnt_ref[...] = jnp.sum(a, axis=1, keepdims=True)


def _hist(ids_tc):
    return pl.pallas_call(
        _hist_body,
        out_shape=jax.ShapeDtypeStruct((B, 1), jnp.float32),
    )(ids_tc)


def _mlp_body(sums_ref, cnt_ref, u_ref, W1_ref, b1_ref, W2_ref, b2_ref,
              out_ref):
    seg = sums_ref[0] + sums_ref[1]
    cnt = cnt_ref[...]
    mean = seg / jnp.maximum(cnt, 1.0)
    u = u_ref[...]
    h = (lax.dot_general(u, W1_ref[0:G, :], (((1,), (0,)), ((), ())),
                         preferred_element_type=jnp.float32)
         + lax.dot_general(mean, W1_ref[G:G + D, :], (((1,), (0,)), ((), ())),
                           preferred_element_type=jnp.float32)
         + b1_ref[...])
    h = jnp.maximum(h, 0.0)
    out_ref[...] = (lax.dot_general(h, W2_ref[...], (((1,), (0,)), ((), ())),
                                    preferred_element_type=jnp.float32)
                    + b2_ref[...] + u)


def _mlp(sums, cnt, u, W1, b1, W2, b2):
    return pl.pallas_call(
        _mlp_body,
        out_shape=jax.ShapeDtypeStruct((B, G), jnp.float32),
    )(sums, cnt, u, W1, b1.reshape(1, H), W2, b2.reshape(1, G))


@jax.jit
def kernel(x, edge_index, edge_attr, u, batch, W1, b1, W2, b2):
    ids = batch.astype(jnp.int32)
    pad_ids = B + (jnp.arange(NPAD - N, dtype=jnp.int32) & 15)
    ids_flat = jnp.concatenate([ids, pad_ids])
    ids_tc = ids_flat.reshape(HR, HC)
    zeros16 = jnp.zeros((16, D), jnp.float32)
    cnt = _hist(ids_tc)
    sums = _seg_reduce(x, ids_flat, zeros16)
    return _mlp(sums, cnt, u, W1, b1, W2, b2)


# speedup vs baseline: 1.0338x; 1.0076x over previous
import numpy as np
import jax
import jax.numpy as jnp
from jax import lax
from jax.experimental import pallas as pl
from jax.experimental.pallas import tpu as pltpu
from jax.experimental.pallas import tpu_sc as plsc

N = 10000
D = 128
B = 256
G = 128
H = 128

NC = 2
NS = 16
NW = NC * NS
ROWS_W = 320
LAST_ROWS = N - (NW - 1) * ROWS_W
NPAD = NW * ROWS_W
ACC_ROWS = 272
HR, HC = 20, 512


def _seg_kernel_body(x_hbm, ids_hbm, zeros_hbm, sums_hbm,
                     rows_v, idsA, idsB, acc_sh, sem_rows, sem_ids, sem_zero):
    c = lax.axis_index("c")
    s = lax.axis_index("s")
    w = c * NS + s
    base = w * ROWS_W

    @pl.when(w < NW - 1)
    def _():
        pltpu.async_copy(x_hbm.at[pl.ds(base, ROWS_W)],
                         rows_v.at[pl.ds(0, ROWS_W)], sem_rows)

    @pl.when(w == NW - 1)
    def _():
        pltpu.async_copy(x_hbm.at[pl.ds(base, LAST_ROWS)],
                         rows_v.at[pl.ds(0, LAST_ROWS)], sem_rows)

    pltpu.async_copy(ids_hbm.at[pl.ds(base, 128)], idsA.at[0], sem_ids)
    pltpu.async_copy(ids_hbm.at[pl.ds(base + 128, 128)], idsA.at[1], sem_ids)
    pltpu.async_copy(ids_hbm.at[pl.ds(base + 256, 64)], idsB, sem_ids)
    pltpu.async_copy(zeros_hbm, acc_sh.at[pl.ds(s * 16, 16)], sem_zero)

    pltpu.make_async_copy(ids_hbm.at[pl.ds(base, 128)], idsA.at[0],
                          sem_ids).wait()
    pltpu.make_async_copy(ids_hbm.at[pl.ds(base + 128, 128)], idsA.at[1],
                          sem_ids).wait()
    pltpu.make_async_copy(ids_hbm.at[pl.ds(base + 256, 64)], idsB,
                          sem_ids).wait()
    pltpu.make_async_copy(zeros_hbm, acc_sh.at[pl.ds(s * 16, 16)],
                          sem_zero).wait()

    plsc.subcore_barrier()

    @pl.when(w < NW - 1)
    def _():
        pltpu.make_async_copy(x_hbm.at[pl.ds(base, ROWS_W)],
                              rows_v.at[pl.ds(0, ROWS_W)], sem_rows).wait()

    @pl.when(w == NW - 1)
    def _():
        pltpu.make_async_copy(x_hbm.at[pl.ds(base, LAST_ROWS)],
                              rows_v.at[pl.ds(0, LAST_ROWS)], sem_rows).wait()

    pltpu.sync_copy(rows_v.at[pl.ds(0, 128)], acc_sh.at[idsA.at[0]], add=True)
    pltpu.sync_copy(rows_v.at[pl.ds(128, 128)], acc_sh.at[idsA.at[1]],
                    add=True)
    pltpu.sync_copy(rows_v.at[pl.ds(256, 64)], acc_sh.at[idsB], add=True)

    plsc.subcore_barrier()

    @pl.when(s == 0)
    def _():
        pltpu.sync_copy(acc_sh.at[pl.ds(0, B)], sums_hbm.at[c])


_seg_reduce = pl.kernel(
    _seg_kernel_body,
    out_type=jax.ShapeDtypeStruct((NC, B, D), jnp.float32),
    mesh=plsc.VectorSubcoreMesh(core_axis_name="c", subcore_axis_name="s"),
    scratch_types=[
        pltpu.VMEM((ROWS_W, D), jnp.float32),
        pltpu.VMEM((2, 128), jnp.int32),
        pltpu.VMEM((64,), jnp.int32),
        pltpu.VMEM_SHARED((ACC_ROWS, D), jnp.float32),
        pltpu.SemaphoreType.DMA,
        pltpu.SemaphoreType.DMA,
        pltpu.SemaphoreType.DMA,
    ],
)


def _hist_body(ids_ref, cnt_ref):
    segs = lax.broadcasted_iota(jnp.int32, (B, 1), 0)
    ids = ids_ref[...].reshape(HR, HC)

    acc = jnp.zeros((B, HC), jnp.float32)
    for r in range(HR):
        row = ids[r:r + 1, :]
        acc = acc + (row == segs).astype(jnp.float32)
    a = (acc[:, 0:128] + acc[:, 128:256]
         + acc[:, 256:384] + acc[:, 384:512])
    cnt_ref[...] = jnp.sum(a, axis=1, keepdims=True)


def _hist(ids_flat):
    return pl.pallas_call(
        _hist_body,
        out_shape=jax.ShapeDtypeStruct((B, 1), jnp.float32),
    )(ids_flat)


def _mlp_body(sums_ref, cnt_ref, u_ref, W1_ref, b1_ref, W2_ref, b2_ref,
              out_ref):
    seg = sums_ref[0] + sums_ref[1]
    cnt = cnt_ref[...]
    mean = seg / jnp.maximum(cnt, 1.0)
    u = u_ref[...]
    h = (lax.dot_general(u, W1_ref[0:G, :], (((1,), (0,)), ((), ())),
                         preferred_element_type=jnp.float32)
         + lax.dot_general(mean, W1_ref[G:G + D, :], (((1,), (0,)), ((), ())),
                           preferred_element_type=jnp.float32)
         + b1_ref[...])
    h = jnp.maximum(h, 0.0)
    out_ref[...] = (lax.dot_general(h, W2_ref[...], (((1,), (0,)), ((), ())),
                                    preferred_element_type=jnp.float32)
                    + b2_ref[...] + u)


def _mlp(sums, cnt, u, W1, b1, W2, b2):
    return pl.pallas_call(
        _mlp_body,
        out_shape=jax.ShapeDtypeStruct((B, G), jnp.float32),
    )(sums, cnt, u, W1, b1.reshape(1, H), W2, b2.reshape(1, G))


_PAD_IDS = np.asarray(B + (np.arange(NPAD - N) & 15), dtype=np.int32)
_ZEROS16 = np.zeros((16, D), dtype=np.float32)


@jax.jit
def kernel(x, edge_index, edge_attr, u, batch, W1, b1, W2, b2):
    ids_flat = jnp.concatenate([batch.astype(jnp.int32), _PAD_IDS])
    cnt = _hist(ids_flat)
    sums = _seg_reduce(x, ids_flat, _ZEROS16)
    return _mlp(sums, cnt, u, W1, b1, W2, b2)
